# baseline (speedup 1.0000x reference)
import functools

import jax
import jax.numpy as jnp
import numpy as np
from jax import lax
from jax.experimental import pallas as pl
from jax.experimental.pallas import tpu as pltpu
from jax.experimental.pallas import tpu_sc as plsc

_N = 10000
_DEG = 32
_D = 128
_B = 8192
_LANES = 16
_PKW = _D // 2


def _sc_gather_pool(nodes, adj_cat, feat_table, feat_bf):
    info = plsc.get_sparse_core_info()
    nc, ns = info.num_cores, info.num_subcores
    nw = nc * ns
    chunk = _B // nw

    mesh = plsc.VectorSubcoreMesh(core_axis_name="c", subcore_axis_name="s")

    nbuf = 8

    @functools.partial(
        pl.kernel,
        mesh=mesh,
        compiler_params=pltpu.CompilerParams(
            use_tc_tiling_on_sc=False, needs_layout_passes=False),
        out_type=[
            jax.ShapeDtypeStruct((_B, _D), jnp.float32),
            jax.ShapeDtypeStruct((_B, _D), jnp.float32),
            jax.ShapeDtypeStruct((_B, _D), jnp.float32),
        ],
        scratch_types=[
            pltpu.VMEM((chunk,), jnp.int32),
            pltpu.VMEM((chunk, 2 * _DEG), jnp.int32),
            pltpu.VMEM((chunk, _D), jnp.float32),
            pltpu.VMEM((chunk, _D), jnp.float32),
        ] + [pltpu.VMEM((2 * _DEG, _D), jnp.bfloat16) for _ in range(nbuf)]
          + [pltpu.SemaphoreType.DMA for _ in range(nbuf + 1)],
    )
    def sc_kernel(nodes_h, adjc_h, feat_h, featb_h, self_o, n0_o, n1_o,
                  idx_v, adj_v, out0_v, out1_v, *bufsem):
        bufs = bufsem[:nbuf]
        sems = bufsem[nbuf:2 * nbuf]
        sem = bufsem[2 * nbuf]
        outs = (out0_v, out1_v)

        wid = lax.axis_index("s") * nc + lax.axis_index("c")
        base = wid * chunk

        pltpu.sync_copy(nodes_h.at[pl.ds(base, chunk)], idx_v)

        nchunk = _D // _LANES
        cp0 = pltpu.async_copy(adjc_h.at[idx_v], adj_v, sem)
        cp1 = pltpu.async_copy(feat_h.at[idx_v], out0_v, sem)
        cp0.wait()
        cp1.wait()
        pltpu.sync_copy(out0_v, self_o.at[pl.ds(base, chunk)])

        def issue(q, b):
            pltpu.async_copy(
                featb_h.at[adj_v.at[q, pl.ds(0, 2 * _DEG)]], bufs[b], sems[b])

        for b in range(nbuf):
            issue(b, b)

        def pool_rows(buf, r):
            def jbody(jj, accs):
                row0 = r * _DEG + jj * 16
                for dj in range(16):
                    new = []
                    for g in range(nchunk // 2):
                        w = plsc.bitcast(
                            buf[row0 + dj, pl.ds(g * 2 * _LANES, 2 * _LANES)],
                            jnp.int32)
                        lo = plsc.bitcast(w << 16, jnp.float32)
                        hi = plsc.bitcast(w, jnp.float32)
                        new.append(accs[2 * g] + lo)
                        new.append(accs[2 * g + 1] + hi)
                    accs = tuple(new)
                return accs
            zero = jnp.zeros((_LANES,), jnp.float32)
            return lax.fori_loop(
                0, _DEG // 16, jbody, tuple(zero for _ in range(nchunk)))

        def body(g, _):
            for b in range(nbuf):
                q = g * nbuf + b
                pltpu.make_async_copy(
                    featb_h.at[adj_v.at[0, pl.ds(0, 2 * _DEG)]],
                    bufs[b], sems[b]).wait()
                for r in range(2):
                    accs = pool_rows(bufs[b], r)
                    for c in range(nchunk):
                        outs[r][q, pl.ds(c * _LANES, _LANES)] = accs[c]
                issue(jnp.minimum(q + nbuf, chunk - 1), b)
            return 0

        lax.fori_loop(0, chunk // nbuf, body, 0)

        for b in range(nbuf):
            pltpu.make_async_copy(
                featb_h.at[adj_v.at[0, pl.ds(0, 2 * _DEG)]],
                bufs[b], sems[b]).wait()

        pltpu.sync_copy(out0_v, n0_o.at[pl.ds(base, chunk)])
        pltpu.sync_copy(out1_v, n1_o.at[pl.ds(base, chunk)])

    return sc_kernel(nodes, adj_cat, feat_table, feat_bf)


def _mlp_body(xs, m0, m1, w1a, w1b, w1c, b1, w2, b2, out):
    h = jnp.dot(xs[:], w1a[:], preferred_element_type=jnp.float32)
    h = h + jnp.dot(m0[:], w1b[:], preferred_element_type=jnp.float32)
    h = h + jnp.dot(m1[:], w1c[:], preferred_element_type=jnp.float32)
    h = jnp.tanh(h + b1[:])
    out[:] = jnp.dot(h, w2[:], preferred_element_type=jnp.float32) + b2[:]


def _tc_mlp(self_f, m0, m1, W1a, W1b, W1c, b1, W2, b2):
    blk = 512
    grid = (_B // blk,)
    row_spec = pl.BlockSpec((blk, _D), lambda i: (i, 0))
    full = lambda shape: pl.BlockSpec(shape, lambda i: (0,) * len(shape))
    return pl.pallas_call(
        _mlp_body,
        grid=grid,
        in_specs=[
            row_spec, row_spec, row_spec,
            full((_D, _D)), full((_D, _D)), full((_D, _D)),
            full((1, _D)), full((_D, _D)), full((1, _D)),
        ],
        out_specs=row_spec,
        out_shape=jax.ShapeDtypeStruct((_B, _D), jnp.float32),
    )(self_f, m0, m1, W1a, W1b, W1c,
      b1.reshape(1, _D), W2, b2.reshape(1, _D))


_UNPACK_PERM = np.concatenate([
    np.concatenate([g * 32 + np.arange(0, 32, 2), g * 32 + np.arange(1, 32, 2)])
    for g in range(_D // 32)])


def kernel(nodes, adj_0, adj_1, feat_table, W1, b1, W2, b2):
    adj_cat = jnp.concatenate([adj_0, adj_1], axis=1)
    feat_bf = feat_table.astype(jnp.bfloat16)
    self_f, m0, m1 = _sc_gather_pool(nodes, adj_cat, feat_table, feat_bf)
    perm = jnp.asarray(_UNPACK_PERM)
    return _tc_mlp(self_f, m0, m1,
                   W1[0:_D],
                   W1[_D:2 * _D][perm] * (1.0 / _DEG),
                   W1[2 * _D:3 * _D][perm] * (1.0 / _DEG),
                   b1, W2, b2)


# speedup vs baseline: 1.0646x; 1.0646x over previous
import functools

import jax
import jax.numpy as jnp
import numpy as np
from jax import lax
from jax.experimental import pallas as pl
from jax.experimental.pallas import tpu as pltpu
from jax.experimental.pallas import tpu_sc as plsc

_N = 10000
_DEG = 32
_D = 128
_B = 8192
_LANES = 16
_PKW = _D // 2


def _sc_gather_pool(nodes, adj_cat, feat_table, feat_bf):
    info = plsc.get_sparse_core_info()
    nc, ns = info.num_cores, info.num_subcores
    nw = nc * ns
    chunk = _B // nw

    mesh = plsc.VectorSubcoreMesh(core_axis_name="c", subcore_axis_name="s")

    nbuf = 8

    @functools.partial(
        pl.kernel,
        mesh=mesh,
        compiler_params=pltpu.CompilerParams(
            use_tc_tiling_on_sc=False, needs_layout_passes=False),
        out_type=[
            jax.ShapeDtypeStruct((_B, _D), jnp.float32),
            jax.ShapeDtypeStruct((_B, _D), jnp.float32),
            jax.ShapeDtypeStruct((_B, _D), jnp.float32),
        ],
        scratch_types=[
            pltpu.VMEM((chunk,), jnp.int32),
            pltpu.VMEM((chunk, 2 * _DEG), jnp.int32),
            pltpu.VMEM((chunk, _D), jnp.float32),
            pltpu.VMEM((chunk, _D), jnp.float32),
        ] + [pltpu.VMEM((2 * _DEG, _D), jnp.bfloat16) for _ in range(nbuf)]
          + [pltpu.SemaphoreType.DMA for _ in range(nbuf + 1)],
    )
    def sc_kernel(nodes_h, adjc_h, feat_h, featb_h, self_o, n0_o, n1_o,
                  idx_v, adj_v, out0_v, out1_v, *bufsem):
        bufs = bufsem[:nbuf]
        sems = bufsem[nbuf:2 * nbuf]
        sem = bufsem[2 * nbuf]
        outs = (out0_v, out1_v)

        wid = lax.axis_index("s") * nc + lax.axis_index("c")
        base = wid * chunk

        pltpu.sync_copy(nodes_h.at[pl.ds(base, chunk)], idx_v)

        nchunk = _D // _LANES
        cp0 = pltpu.async_copy(adjc_h.at[idx_v], adj_v, sem)
        cp1 = pltpu.async_copy(feat_h.at[idx_v], out0_v, sem)
        cp0.wait()
        cp1.wait()
        pltpu.sync_copy(out0_v, self_o.at[pl.ds(base, chunk)])

        def issue(q, b):
            pltpu.async_copy(
                featb_h.at[adj_v.at[q, pl.ds(0, 2 * _DEG)]], bufs[b], sems[b])

        for b in range(nbuf):
            issue(b, b)

        def pool_rows(buf, r):
            def jbody(jj, accs):
                row0 = r * _DEG + jj * 16
                for dj in range(16):
                    new = []
                    for g in range(nchunk // 2):
                        w = plsc.bitcast(
                            buf[row0 + dj, pl.ds(g * 2 * _LANES, 2 * _LANES)],
                            jnp.int32)
                        lo = plsc.bitcast(w << 16, jnp.float32)
                        hi = plsc.bitcast(w, jnp.float32)
                        new.append(accs[2 * g] + lo)
                        new.append(accs[2 * g + 1] + hi)
                    accs = tuple(new)
                return accs
            zero = jnp.zeros((_LANES,), jnp.float32)
            return lax.fori_loop(
                0, _DEG // 16, jbody, tuple(zero for _ in range(nchunk)))

        def body(g, _):
            for b in range(nbuf):
                q = g * nbuf + b
                pltpu.make_async_copy(
                    featb_h.at[adj_v.at[0, pl.ds(0, 2 * _DEG)]],
                    bufs[b], sems[b]).wait()
                for r in range(2):
                    accs = pool_rows(bufs[b], r)
                    for c in range(nchunk):
                        outs[r][q, pl.ds(c * _LANES, _LANES)] = accs[c]
                issue(jnp.minimum(q + nbuf, chunk - 1), b)
            return 0

        lax.fori_loop(0, chunk // nbuf, body, 0)

        for b in range(nbuf):
            pltpu.make_async_copy(
                featb_h.at[adj_v.at[0, pl.ds(0, 2 * _DEG)]],
                bufs[b], sems[b]).wait()

        pltpu.sync_copy(out0_v, n0_o.at[pl.ds(base, chunk)])
        pltpu.sync_copy(out1_v, n1_o.at[pl.ds(base, chunk)])

    return sc_kernel(nodes, adj_cat, feat_table, feat_bf)


def _mlp_body(xs, m0, m1, w1a, w1b, w1c, b1, w2, b2, out):
    h = jnp.dot(xs[:], w1a[:], preferred_element_type=jnp.float32)
    h = h + jnp.dot(m0[:], w1b[:], preferred_element_type=jnp.float32)
    h = h + jnp.dot(m1[:], w1c[:], preferred_element_type=jnp.float32)
    h = jnp.tanh(h + b1[:])
    out[:] = jnp.dot(h, w2[:], preferred_element_type=jnp.float32) + b2[:]


def _tc_mlp(self_f, m0, m1, W1a, W1b, W1c, b1, W2, b2):
    blk = 2048
    grid = (_B // blk,)
    row_spec = pl.BlockSpec((blk, _D), lambda i: (i, 0))
    full = lambda shape: pl.BlockSpec(shape, lambda i: (0,) * len(shape))
    return pl.pallas_call(
        _mlp_body,
        grid=grid,
        in_specs=[
            row_spec, row_spec, row_spec,
            full((_D, _D)), full((_D, _D)), full((_D, _D)),
            full((1, _D)), full((_D, _D)), full((1, _D)),
        ],
        out_specs=row_spec,
        out_shape=jax.ShapeDtypeStruct((_B, _D), jnp.float32),
    )(self_f, m0, m1, W1a, W1b, W1c,
      b1.reshape(1, _D), W2, b2.reshape(1, _D))


_UNPACK_PERM = np.concatenate([
    np.concatenate([g * 32 + np.arange(0, 32, 2), g * 32 + np.arange(1, 32, 2)])
    for g in range(_D // 32)])


def kernel(nodes, adj_0, adj_1, feat_table, W1, b1, W2, b2):
    adj_cat = jnp.concatenate([adj_0, adj_1], axis=1)
    feat_bf = feat_table.astype(jnp.bfloat16)
    self_f, m0, m1 = _sc_gather_pool(nodes, adj_cat, feat_table, feat_bf)
    perm = jnp.asarray(_UNPACK_PERM)
    return _tc_mlp(self_f, m0, m1,
                   W1[0:_D],
                   W1[_D:2 * _D][perm] * (1.0 / _DEG),
                   W1[2 * _D:3 * _D][perm] * (1.0 / _DEG),
                   b1, W2, b2)


# speedup vs baseline: 1.0730x; 1.0079x over previous
import functools

import jax
import jax.numpy as jnp
import numpy as np
from jax import lax
from jax.experimental import pallas as pl
from jax.experimental.pallas import tpu as pltpu
from jax.experimental.pallas import tpu_sc as plsc

_N = 10000
_DEG = 32
_D = 128
_B = 8192
_LANES = 16
_PKW = _D // 2


def _sc_gather_pool(nodes, adj_cat, feat_table, feat_bf):
    info = plsc.get_sparse_core_info()
    nc, ns = info.num_cores, info.num_subcores
    nw = nc * ns
    chunk = _B // nw

    mesh = plsc.VectorSubcoreMesh(core_axis_name="c", subcore_axis_name="s")

    nbuf = 8

    @functools.partial(
        pl.kernel,
        mesh=mesh,
        compiler_params=pltpu.CompilerParams(
            use_tc_tiling_on_sc=False, needs_layout_passes=False),
        out_type=[
            jax.ShapeDtypeStruct((_B, _D), jnp.float32),
            jax.ShapeDtypeStruct((_B, _D), jnp.float32),
            jax.ShapeDtypeStruct((_B, _D), jnp.float32),
        ],
        scratch_types=[
            pltpu.VMEM((chunk,), jnp.int32),
            pltpu.VMEM((chunk, 2 * _DEG), jnp.int32),
            pltpu.VMEM((chunk, _D), jnp.float32),
            pltpu.VMEM((chunk, _D), jnp.float32),
        ] + [pltpu.VMEM((2 * _DEG, _D), jnp.bfloat16) for _ in range(nbuf)]
          + [pltpu.SemaphoreType.DMA for _ in range(nbuf + 1)],
    )
    def sc_kernel(nodes_h, adjc_h, feat_h, featb_h, self_o, n0_o, n1_o,
                  idx_v, adj_v, out0_v, out1_v, *bufsem):
        bufs = bufsem[:nbuf]
        sems = bufsem[nbuf:2 * nbuf]
        sem = bufsem[2 * nbuf]
        outs = (out0_v, out1_v)

        wid = lax.axis_index("s") * nc + lax.axis_index("c")
        base = wid * chunk

        pltpu.sync_copy(nodes_h.at[pl.ds(base, chunk)], idx_v)

        nchunk = _D // _LANES
        cp0 = pltpu.async_copy(adjc_h.at[idx_v], adj_v, sem)
        cp1 = pltpu.async_copy(feat_h.at[idx_v], out0_v, sem)
        cp0.wait()
        cp1.wait()
        pltpu.sync_copy(out0_v, self_o.at[pl.ds(base, chunk)])

        def issue(q, b):
            pltpu.async_copy(
                featb_h.at[adj_v.at[q, pl.ds(0, 2 * _DEG)]], bufs[b], sems[b])

        for b in range(nbuf):
            issue(b, b)

        def pool_rows(buf, r):
            def jbody(jj, accs):
                row0 = r * _DEG + jj * 16
                for dj in range(16):
                    new = []
                    for g in range(nchunk // 2):
                        w = plsc.bitcast(
                            buf[row0 + dj, pl.ds(g * 2 * _LANES, 2 * _LANES)],
                            jnp.int32)
                        lo = plsc.bitcast(w << 16, jnp.float32)
                        hi = plsc.bitcast(w, jnp.float32)
                        new.append(accs[2 * g] + lo)
                        new.append(accs[2 * g + 1] + hi)
                    accs = tuple(new)
                return accs
            zero = jnp.zeros((_LANES,), jnp.float32)
            return lax.fori_loop(
                0, _DEG // 16, jbody, tuple(zero for _ in range(nchunk)))

        def body(g, _):
            for b in range(nbuf):
                q = g * nbuf + b
                pltpu.make_async_copy(
                    featb_h.at[adj_v.at[0, pl.ds(0, 2 * _DEG)]],
                    bufs[b], sems[b]).wait()
                for r in range(2):
                    accs = pool_rows(bufs[b], r)
                    for c in range(nchunk):
                        outs[r][q, pl.ds(c * _LANES, _LANES)] = accs[c]
                issue(jnp.minimum(q + nbuf, chunk - 1), b)
            return 0

        lax.fori_loop(0, chunk // nbuf, body, 0)

        for b in range(nbuf):
            pltpu.make_async_copy(
                featb_h.at[adj_v.at[0, pl.ds(0, 2 * _DEG)]],
                bufs[b], sems[b]).wait()

        pltpu.sync_copy(out0_v, n0_o.at[pl.ds(base, chunk)])
        pltpu.sync_copy(out1_v, n1_o.at[pl.ds(base, chunk)])

    return sc_kernel(nodes, adj_cat, feat_table, feat_bf)


def _mlp_body(xs, m0, m1, w1a, w1b, w1c, b1, w2, b2, out):
    h = jnp.dot(xs[:], w1a[:], preferred_element_type=jnp.float32)
    h = h + jnp.dot(m0[:], w1b[:], preferred_element_type=jnp.float32)
    h = h + jnp.dot(m1[:], w1c[:], preferred_element_type=jnp.float32)
    h = jnp.tanh(h + b1[:])
    out[:] = jnp.dot(h, w2[:], preferred_element_type=jnp.float32) + b2[:]


def _tc_mlp(self_f, m0, m1, W1a, W1b, W1c, b1, W2, b2):
    blk = 4096
    grid = (_B // blk,)
    row_spec = pl.BlockSpec((blk, _D), lambda i: (i, 0))
    full = lambda shape: pl.BlockSpec(shape, lambda i: (0,) * len(shape))
    return pl.pallas_call(
        _mlp_body,
        grid=grid,
        in_specs=[
            row_spec, row_spec, row_spec,
            full((_D, _D)), full((_D, _D)), full((_D, _D)),
            full((1, _D)), full((_D, _D)), full((1, _D)),
        ],
        out_specs=row_spec,
        out_shape=jax.ShapeDtypeStruct((_B, _D), jnp.float32),
    )(self_f, m0, m1, W1a, W1b, W1c,
      b1.reshape(1, _D), W2, b2.reshape(1, _D))


_UNPACK_PERM = np.concatenate([
    np.concatenate([g * 32 + np.arange(0, 32, 2), g * 32 + np.arange(1, 32, 2)])
    for g in range(_D // 32)])


def kernel(nodes, adj_0, adj_1, feat_table, W1, b1, W2, b2):
    adj_cat = jnp.concatenate([adj_0, adj_1], axis=1)
    feat_bf = feat_table.astype(jnp.bfloat16)
    self_f, m0, m1 = _sc_gather_pool(nodes, adj_cat, feat_table, feat_bf)
    perm = jnp.asarray(_UNPACK_PERM)
    return _tc_mlp(self_f, m0, m1,
                   W1[0:_D],
                   W1[_D:2 * _D][perm] * (1.0 / _DEG),
                   W1[2 * _D:3 * _D][perm] * (1.0 / _DEG),
                   b1, W2, b2)
